# baseline (speedup 1.0000x reference)
import functools

import jax
import jax.numpy as jnp
from jax import lax
from jax.experimental import pallas as pl
from jax.experimental.pallas import tpu as pltpu
from jax.experimental.pallas import tpu_sc as plsc

LANES = 16


def _sc_worker_count() -> tuple[int, int]:
  try:
    info = plsc.get_sparse_core_info()
    return info.num_cores, info.num_subcores
  except Exception:
    return 2, 16


@functools.lru_cache(maxsize=None)
def _build_pool(batch: int, hist: int, dim: int):
  nc, ns = _sc_worker_count()
  nw = nc * ns
  assert batch % nw == 0
  bpw = batch // nw
  nbuf = 4
  chunk = 128
  assert bpw % chunk == 0 and chunk % nbuf == 0
  nch = bpw // chunk
  ngrp = chunk // nbuf - 1
  split = 128
  rest = hist - split
  assert 0 < rest <= 128 and hist % 8 == 0 and dim % LANES == 0
  nw_row = dim // 2
  nwg = nw_row // LANES
  inv = jnp.float32(1.0 / hist)

  mesh = plsc.VectorSubcoreMesh(core_axis_name="c", subcore_axis_name="s")

  @functools.partial(
      pl.kernel,
      out_type=jax.ShapeDtypeStruct((batch, dim), jnp.float32),
      mesh=mesh,
      scratch_types=[
          pltpu.VMEM((chunk * hist,), jnp.int32),
          pltpu.VMEM((nbuf, hist, nw_row), jnp.int32),
          pltpu.VMEM((bpw, dim), jnp.float32),
          pltpu.SemaphoreType.DMA((nbuf,)),
      ],
      compiler_params=pltpu.CompilerParams(
          use_tc_tiling_on_sc=False, needs_layout_passes=False),
  )
  def pool(x_hbm, table_hbm, out_hbm, idx_v, rows_v, acc_v, sem):
    wid = lax.axis_index("s") * nc + lax.axis_index("c")
    row0 = wid * bpw

    def issue(crow, slot):
      off = crow * hist
      pltpu.async_copy(
          table_hbm.at[idx_v.at[pl.ds(off, split)]],
          rows_v.at[slot, pl.ds(0, split)],
          sem.at[slot],
      )
      pltpu.async_copy(
          table_hbm.at[idx_v.at[pl.ds(off + split, rest)]],
          rows_v.at[slot, pl.ds(split, rest)],
          sem.at[slot],
      )

    def wait(slot):
      pltpu.make_async_copy(
          table_hbm.at[pl.ds(0, split)],
          rows_v.at[slot, pl.ds(0, split)],
          sem.at[slot],
      ).wait()
      pltpu.make_async_copy(
          table_hbm.at[pl.ds(0, rest)],
          rows_v.at[slot, pl.ds(split, rest)],
          sem.at[slot],
      ).wait()

    def reduce(slot, brow):
      r = rows_v.at[slot]
      hi_mask = jnp.full((LANES,), -65536, jnp.int32)

      def step(j, carry):
        out = list(carry)
        for g in range(nwg):
          w = r[j, pl.ds(LANES * g, LANES)]
          out[g] = out[g] + plsc.bitcast(w << 16, jnp.float32)
          out[g + nwg] = out[g + nwg] + plsc.bitcast(w & hi_mask, jnp.float32)
        return tuple(out)

      zeros = (jnp.zeros((LANES,), jnp.float32),) * (2 * nwg)
      acc = pl.loop(0, hist, init_carry=zeros, unroll=8)(step)
      for d in range(2 * nwg):
        acc_v[brow, pl.ds(LANES * d, LANES)] = acc[d] * inv

    def transform(_=None):
      c1 = jnp.full((LANES,), _Q, jnp.int32)
      c2 = jnp.full((LANES,), 2 * _Q, jnp.int32)
      c3 = jnp.full((LANES,), 3 * _Q, jnp.int32)
      one = jnp.full((LANES,), 1, jnp.int32)
      zero = jnp.zeros((LANES,), jnp.int32)
      qc = jnp.full((LANES,), 4 * _Q - 1, jnp.int32)

      def tstep(i):
        v = idx_v[pl.ds(i * LANES, LANES)]
        q = (jnp.where(v >= c1, one, zero)
             + jnp.where(v >= c2, one, zero)
             + jnp.where(v >= c3, one, zero))
        idx_v[pl.ds(i * LANES, LANES)] = (v << 2) - q * qc

      pl.loop(0, chunk * hist // LANES, unroll=8)(tstep)

    def chunk_body(c):
      base = c * chunk
      pltpu.sync_copy(
          x_hbm.at[pl.ds((row0 + base) * hist, chunk * hist)], idx_v
      )
      transform()
      for k in range(nbuf):
        issue(k, k)

      def grp(g):
        for k in range(nbuf):
          j = g * nbuf + k
          wait(k)
          reduce(k, base + j)
          issue(j + nbuf, k)

      pl.loop(0, ngrp)(grp)
      for k in range(nbuf):
        wait(k)
        reduce(k, base + (ngrp * nbuf + k))

    pl.loop(0, nch)(chunk_body)
    pltpu.sync_copy(acc_v, out_hbm.at[pl.ds(row0, bpw)])

  return pool


_PACK_B = 3968
_Q = 249984
_QR = 250048


def _pack_body(t0_ref, t1_ref, t2_ref, t3_ref, o_ref):
  eye = jnp.eye(32, dtype=jnp.float32)
  dn = (((0,), (0,)), ((), ()))
  for q, t_ref in enumerate((t0_ref, t1_ref, t2_ref, t3_ref)):
    lo = lax.bitcast_convert_type(
        lax.dot_general(t_ref[0:32, :], eye, dn,
                        preferred_element_type=jnp.float32), jnp.uint32)
    hi = lax.bitcast_convert_type(
        lax.dot_general(t_ref[32:64, :], eye, dn,
                        preferred_element_type=jnp.float32), jnp.uint32)
    word = (lo >> jnp.uint32(16)) | (hi & jnp.uint32(0xFFFF0000))
    o_ref[:, 32 * q:32 * (q + 1)] = lax.bitcast_convert_type(word, jnp.int32)


@functools.lru_cache(maxsize=None)
def _build_pack(vocab: int, dim: int):
  assert dim == 64 and vocab == 4 * _Q + 64
  grid = (_QR + _PACK_B - 1) // _PACK_B
  off = _Q // _PACK_B
  return pl.pallas_call(
      _pack_body,
      grid=(grid,),
      in_specs=[
          pl.BlockSpec((dim, _PACK_B), lambda i, q=q: (0, i + q * off))
          for q in range(4)
      ],
      out_specs=pl.BlockSpec((_PACK_B, 128), lambda i: (i, 0)),
      out_shape=jax.ShapeDtypeStruct((_QR, 128), jnp.int32),
  )


def _mm_body(p_ref, w_ref, b_ref, o_ref):
  o_ref[...] = (
      jnp.dot(p_ref[...], w_ref[...], preferred_element_type=jnp.float32)
      + b_ref[...]
  )


@functools.lru_cache(maxsize=None)
def _build_linear(batch: int, dim: int, odim: int):
  bm = 2048
  assert batch % bm == 0
  return pl.pallas_call(
      _mm_body,
      grid=(batch // bm,),
      in_specs=[
          pl.BlockSpec((bm, dim), lambda i: (i, 0)),
          pl.BlockSpec((dim, odim), lambda i: (0, 0)),
          pl.BlockSpec((1, odim), lambda i: (0, 0)),
      ],
      out_specs=pl.BlockSpec((bm, odim), lambda i: (i, 0)),
      out_shape=jax.ShapeDtypeStruct((batch, odim), jnp.float32),
  )


def kernel(x, table, W, b):
  batch, hist = x.shape
  vocab, dim = table.shape
  odim = W.shape[1]
  x_flat = jnp.asarray(x, jnp.int32).reshape(batch * hist)
  table_t, x_flat = lax.optimization_barrier((table.T, x_flat))
  table_pk = _build_pack(vocab, dim)(
      table_t, table_t, table_t, table_t).reshape(4 * _QR, dim // 2)
  pooled = _build_pool(batch, hist, dim)(x_flat, table_pk)
  return _build_linear(batch, dim, odim)(pooled, W, b.reshape(1, odim))


# speedup vs baseline: 1.5609x; 1.5609x over previous
import functools

import jax
import jax.numpy as jnp
from jax import lax
from jax.experimental import pallas as pl
from jax.experimental.pallas import tpu as pltpu
from jax.experimental.pallas import tpu_sc as plsc

LANES = 16


def _sc_worker_count() -> tuple[int, int]:
  try:
    info = plsc.get_sparse_core_info()
    return info.num_cores, info.num_subcores
  except Exception:
    return 2, 16


@functools.lru_cache(maxsize=None)
def _build_pool(batch: int, hist: int, dim: int):
  nc, ns = _sc_worker_count()
  nw = nc * ns
  assert batch % nw == 0
  bpw = batch // nw
  nbuf = 4
  chunk = 128
  assert bpw % chunk == 0 and chunk % nbuf == 0
  nch = bpw // chunk
  ngrp = chunk // nbuf - 1
  split = 128
  rest = hist - split
  assert 0 < rest <= 128 and hist % 8 == 0 and dim % LANES == 0
  nw_row = dim // 2
  nwg = nw_row // LANES
  inv = jnp.float32(1.0 / hist)

  mesh = plsc.VectorSubcoreMesh(core_axis_name="c", subcore_axis_name="s")

  @functools.partial(
      pl.kernel,
      out_type=jax.ShapeDtypeStruct((batch, dim), jnp.float32),
      mesh=mesh,
      scratch_types=[
          pltpu.VMEM((chunk * hist,), jnp.int32),
          pltpu.VMEM((nbuf, hist, nw_row), jnp.int32),
          pltpu.VMEM((bpw, dim), jnp.float32),
          pltpu.SemaphoreType.DMA((nbuf,)),
      ],
      compiler_params=pltpu.CompilerParams(
          use_tc_tiling_on_sc=False, needs_layout_passes=False),
  )
  def pool(x_hbm, table_hbm, out_hbm, idx_v, rows_v, acc_v, sem):
    wid = lax.axis_index("s") * nc + lax.axis_index("c")
    row0 = wid * bpw

    def issue(crow, slot):
      off = crow * hist
      pltpu.async_copy(
          table_hbm.at[idx_v.at[pl.ds(off, split)]],
          rows_v.at[slot, pl.ds(0, split)],
          sem.at[slot],
      )
      pltpu.async_copy(
          table_hbm.at[idx_v.at[pl.ds(off + split, rest)]],
          rows_v.at[slot, pl.ds(split, rest)],
          sem.at[slot],
      )

    def wait(slot):
      pltpu.make_async_copy(
          table_hbm.at[pl.ds(0, split)],
          rows_v.at[slot, pl.ds(0, split)],
          sem.at[slot],
      ).wait()
      pltpu.make_async_copy(
          table_hbm.at[pl.ds(0, rest)],
          rows_v.at[slot, pl.ds(split, rest)],
          sem.at[slot],
      ).wait()

    def reduce(slot, brow):
      r = rows_v.at[slot]
      hi_mask = jnp.full((LANES,), -65536, jnp.int32)

      def step(j, carry):
        out = list(carry)
        for g in range(nwg):
          w = r[j, pl.ds(LANES * g, LANES)]
          out[g] = out[g] + plsc.bitcast(w << 16, jnp.float32)
          out[g + nwg] = out[g + nwg] + plsc.bitcast(w & hi_mask, jnp.float32)
        return tuple(out)

      zeros = (jnp.zeros((LANES,), jnp.float32),) * (2 * nwg)
      acc = pl.loop(0, hist, init_carry=zeros, unroll=8)(step)
      for d in range(2 * nwg):
        acc_v[brow, pl.ds(LANES * d, LANES)] = acc[d] * inv

    def transform(_=None):
      c1 = jnp.full((LANES,), _Q, jnp.int32)
      c2 = jnp.full((LANES,), 2 * _Q, jnp.int32)
      c3 = jnp.full((LANES,), 3 * _Q, jnp.int32)
      one = jnp.full((LANES,), 1, jnp.int32)
      zero = jnp.zeros((LANES,), jnp.int32)
      qc = jnp.full((LANES,), 4 * _Q - 1, jnp.int32)

      def tstep(i):
        v = idx_v[pl.ds(i * LANES, LANES)]
        q = (jnp.where(v >= c1, one, zero)
             + jnp.where(v >= c2, one, zero)
             + jnp.where(v >= c3, one, zero))
        idx_v[pl.ds(i * LANES, LANES)] = (v << 2) - q * qc

      pl.loop(0, chunk * hist // LANES, unroll=8)(tstep)

    def chunk_body(c):
      base = c * chunk
      pltpu.sync_copy(
          x_hbm.at[pl.ds((row0 + base) * hist, chunk * hist)], idx_v
      )
      transform()
      for k in range(nbuf):
        issue(k, k)

      def grp(g):
        for k in range(nbuf):
          j = g * nbuf + k
          wait(k)
          reduce(k, base + j)
          issue(j + nbuf, k)

      pl.loop(0, ngrp)(grp)
      for k in range(nbuf):
        wait(k)
        reduce(k, base + (ngrp * nbuf + k))

    pl.loop(0, nch)(chunk_body)
    pltpu.sync_copy(acc_v, out_hbm.at[pl.ds(row0, bpw)])

  return pool


_PACK_B = 3968
_Q = 249984
_QR = 250048


def _pack_body(t0_ref, t1_ref, t2_ref, t3_ref, o_ref):
  t = jnp.concatenate(
      [t0_ref[...], t1_ref[...], t2_ref[...], t3_ref[...]], axis=0)
  r = lax.broadcasted_iota(jnp.int32, (256, 128), 0)
  c = lax.broadcasted_iota(jnp.int32, (256, 128), 1)
  same_q = (r >> 6) == (c >> 5)
  d = r & 63
  cc = c & 31
  one = jnp.float32(1.0)
  zero = jnp.float32(0.0)
  e_lo = jnp.where(same_q & (d == cc), one, zero)
  e_hi = jnp.where(same_q & (d == cc + 32), one, zero)
  dn = (((0,), (0,)), ((), ()))
  lo = lax.bitcast_convert_type(
      lax.dot_general(t, e_lo, dn, preferred_element_type=jnp.float32),
      jnp.uint32)
  hi = lax.bitcast_convert_type(
      lax.dot_general(t, e_hi, dn, preferred_element_type=jnp.float32),
      jnp.uint32)
  word = (lo >> jnp.uint32(16)) | (hi & jnp.uint32(0xFFFF0000))
  o_ref[...] = lax.bitcast_convert_type(word, jnp.int32)


@functools.lru_cache(maxsize=None)
def _build_pack(vocab: int, dim: int):
  assert dim == 64 and vocab == 4 * _Q + 64
  grid = (_QR + _PACK_B - 1) // _PACK_B
  off = _Q // _PACK_B
  return pl.pallas_call(
      _pack_body,
      grid=(grid,),
      in_specs=[
          pl.BlockSpec((dim, _PACK_B), lambda i, q=q: (0, i + q * off))
          for q in range(4)
      ],
      out_specs=pl.BlockSpec((_PACK_B, 128), lambda i: (i, 0)),
      out_shape=jax.ShapeDtypeStruct((_QR, 128), jnp.int32),
  )


def _mm_body(p_ref, w_ref, b_ref, o_ref):
  o_ref[...] = (
      jnp.dot(p_ref[...], w_ref[...], preferred_element_type=jnp.float32)
      + b_ref[...]
  )


@functools.lru_cache(maxsize=None)
def _build_linear(batch: int, dim: int, odim: int):
  bm = 2048
  assert batch % bm == 0
  return pl.pallas_call(
      _mm_body,
      grid=(batch // bm,),
      in_specs=[
          pl.BlockSpec((bm, dim), lambda i: (i, 0)),
          pl.BlockSpec((dim, odim), lambda i: (0, 0)),
          pl.BlockSpec((1, odim), lambda i: (0, 0)),
      ],
      out_specs=pl.BlockSpec((bm, odim), lambda i: (i, 0)),
      out_shape=jax.ShapeDtypeStruct((batch, odim), jnp.float32),
  )


def kernel(x, table, W, b):
  batch, hist = x.shape
  vocab, dim = table.shape
  odim = W.shape[1]
  x_flat = jnp.asarray(x, jnp.int32).reshape(batch * hist)
  table_t, x_flat = lax.optimization_barrier((table.T, x_flat))
  table_pk = _build_pack(vocab, dim)(
      table_t, table_t, table_t, table_t).reshape(4 * _QR, dim // 2)
  pooled = _build_pool(batch, hist, dim)(x_flat, table_pk)
  return _build_linear(batch, dim, odim)(pooled, W, b.reshape(1, odim))


# speedup vs baseline: 1.5684x; 1.0048x over previous
import functools

import jax
import jax.numpy as jnp
from jax import lax
from jax.experimental import pallas as pl
from jax.experimental.pallas import tpu as pltpu
from jax.experimental.pallas import tpu_sc as plsc

LANES = 16


def _sc_worker_count() -> tuple[int, int]:
  try:
    info = plsc.get_sparse_core_info()
    return info.num_cores, info.num_subcores
  except Exception:
    return 2, 16


@functools.lru_cache(maxsize=None)
def _build_pool(batch: int, hist: int, dim: int):
  nc, ns = _sc_worker_count()
  nw = nc * ns
  assert batch % nw == 0
  bpw = batch // nw
  nbuf = 8
  chunk = 128
  assert bpw % chunk == 0 and chunk % nbuf == 0
  nch = bpw // chunk
  ngrp = chunk // nbuf - 1
  split = 128
  rest = hist - split
  assert 0 < rest <= 128 and hist % 8 == 0 and dim % LANES == 0
  nw_row = dim // 2
  nwg = nw_row // LANES
  inv = jnp.float32(1.0 / hist)

  mesh = plsc.VectorSubcoreMesh(core_axis_name="c", subcore_axis_name="s")

  @functools.partial(
      pl.kernel,
      out_type=jax.ShapeDtypeStruct((batch, dim), jnp.float32),
      mesh=mesh,
      scratch_types=[
          pltpu.VMEM((chunk * hist,), jnp.int32),
          pltpu.VMEM((nbuf, hist, nw_row), jnp.int32),
          pltpu.VMEM((bpw, dim), jnp.float32),
          pltpu.SemaphoreType.DMA((nbuf,)),
      ],
      compiler_params=pltpu.CompilerParams(
          use_tc_tiling_on_sc=False, needs_layout_passes=False),
  )
  def pool(x_hbm, table_hbm, out_hbm, idx_v, rows_v, acc_v, sem):
    wid = lax.axis_index("s") * nc + lax.axis_index("c")
    row0 = wid * bpw

    def issue(crow, slot):
      off = crow * hist
      pltpu.async_copy(
          table_hbm.at[idx_v.at[pl.ds(off, split)]],
          rows_v.at[slot, pl.ds(0, split)],
          sem.at[slot],
      )
      pltpu.async_copy(
          table_hbm.at[idx_v.at[pl.ds(off + split, rest)]],
          rows_v.at[slot, pl.ds(split, rest)],
          sem.at[slot],
      )

    def wait(slot):
      pltpu.make_async_copy(
          table_hbm.at[pl.ds(0, split)],
          rows_v.at[slot, pl.ds(0, split)],
          sem.at[slot],
      ).wait()
      pltpu.make_async_copy(
          table_hbm.at[pl.ds(0, rest)],
          rows_v.at[slot, pl.ds(split, rest)],
          sem.at[slot],
      ).wait()

    def reduce(slot, brow):
      r = rows_v.at[slot]
      hi_mask = jnp.full((LANES,), -65536, jnp.int32)

      def step(j, carry):
        out = list(carry)
        for g in range(nwg):
          w = r[j, pl.ds(LANES * g, LANES)]
          out[g] = out[g] + plsc.bitcast(w << 16, jnp.float32)
          out[g + nwg] = out[g + nwg] + plsc.bitcast(w & hi_mask, jnp.float32)
        return tuple(out)

      zeros = (jnp.zeros((LANES,), jnp.float32),) * (2 * nwg)
      acc = pl.loop(0, hist, init_carry=zeros, unroll=8)(step)
      for d in range(2 * nwg):
        acc_v[brow, pl.ds(LANES * d, LANES)] = acc[d] * inv

    def transform(_=None):
      c1 = jnp.full((LANES,), _Q, jnp.int32)
      c2 = jnp.full((LANES,), 2 * _Q, jnp.int32)
      c3 = jnp.full((LANES,), 3 * _Q, jnp.int32)
      one = jnp.full((LANES,), 1, jnp.int32)
      zero = jnp.zeros((LANES,), jnp.int32)
      qc = jnp.full((LANES,), 4 * _Q - 1, jnp.int32)

      def tstep(i):
        v = idx_v[pl.ds(i * LANES, LANES)]
        q = (jnp.where(v >= c1, one, zero)
             + jnp.where(v >= c2, one, zero)
             + jnp.where(v >= c3, one, zero))
        idx_v[pl.ds(i * LANES, LANES)] = (v << 2) - q * qc

      pl.loop(0, chunk * hist // LANES, unroll=8)(tstep)

    def chunk_body(c):
      base = c * chunk
      pltpu.sync_copy(
          x_hbm.at[pl.ds((row0 + base) * hist, chunk * hist)], idx_v
      )
      transform()
      for k in range(nbuf):
        issue(k, k)

      def grp(g):
        for k in range(nbuf):
          j = g * nbuf + k
          wait(k)
          reduce(k, base + j)
          issue(j + nbuf, k)

      pl.loop(0, ngrp)(grp)
      for k in range(nbuf):
        wait(k)
        reduce(k, base + (ngrp * nbuf + k))

    pl.loop(0, nch)(chunk_body)
    pltpu.sync_copy(acc_v, out_hbm.at[pl.ds(row0, bpw)])

  return pool


_PACK_B = 3968
_Q = 249984
_QR = 250048


def _pack_body(t0_ref, t1_ref, t2_ref, t3_ref, o_ref):
  t = jnp.concatenate(
      [t0_ref[...], t1_ref[...], t2_ref[...], t3_ref[...]], axis=0)
  r = lax.broadcasted_iota(jnp.int32, (256, 128), 0)
  c = lax.broadcasted_iota(jnp.int32, (256, 128), 1)
  same_q = (r >> 6) == (c >> 5)
  d = r & 63
  cc = c & 31
  one = jnp.float32(1.0)
  zero = jnp.float32(0.0)
  e_lo = jnp.where(same_q & (d == cc), one, zero)
  e_hi = jnp.where(same_q & (d == cc + 32), one, zero)
  dn = (((0,), (0,)), ((), ()))
  lo = lax.bitcast_convert_type(
      lax.dot_general(t, e_lo, dn, preferred_element_type=jnp.float32),
      jnp.uint32)
  hi = lax.bitcast_convert_type(
      lax.dot_general(t, e_hi, dn, preferred_element_type=jnp.float32),
      jnp.uint32)
  word = (lo >> jnp.uint32(16)) | (hi & jnp.uint32(0xFFFF0000))
  o_ref[...] = lax.bitcast_convert_type(word, jnp.int32)


@functools.lru_cache(maxsize=None)
def _build_pack(vocab: int, dim: int):
  assert dim == 64 and vocab == 4 * _Q + 64
  grid = (_QR + _PACK_B - 1) // _PACK_B
  off = _Q // _PACK_B
  return pl.pallas_call(
      _pack_body,
      grid=(grid,),
      in_specs=[
          pl.BlockSpec((dim, _PACK_B), lambda i, q=q: (0, i + q * off))
          for q in range(4)
      ],
      out_specs=pl.BlockSpec((_PACK_B, 128), lambda i: (i, 0)),
      out_shape=jax.ShapeDtypeStruct((_QR, 128), jnp.int32),
  )


def _mm_body(p_ref, w_ref, b_ref, o_ref):
  o_ref[...] = (
      jnp.dot(p_ref[...], w_ref[...], preferred_element_type=jnp.float32)
      + b_ref[...]
  )


@functools.lru_cache(maxsize=None)
def _build_linear(batch: int, dim: int, odim: int):
  bm = 2048
  assert batch % bm == 0
  return pl.pallas_call(
      _mm_body,
      grid=(batch // bm,),
      in_specs=[
          pl.BlockSpec((bm, dim), lambda i: (i, 0)),
          pl.BlockSpec((dim, odim), lambda i: (0, 0)),
          pl.BlockSpec((1, odim), lambda i: (0, 0)),
      ],
      out_specs=pl.BlockSpec((bm, odim), lambda i: (i, 0)),
      out_shape=jax.ShapeDtypeStruct((batch, odim), jnp.float32),
  )


def kernel(x, table, W, b):
  batch, hist = x.shape
  vocab, dim = table.shape
  odim = W.shape[1]
  x_flat = jnp.asarray(x, jnp.int32).reshape(batch * hist)
  table_t, x_flat = lax.optimization_barrier((table.T, x_flat))
  table_pk = _build_pack(vocab, dim)(
      table_t, table_t, table_t, table_t).reshape(4 * _QR, dim // 2)
  pooled = _build_pool(batch, hist, dim)(x_flat, table_pk)
  return _build_linear(batch, dim, odim)(pooled, W, b.reshape(1, odim))
